# baseline (device time: 71412 ns/iter reference)
import jax
import jax.numpy as jnp
from jax import lax
from jax.experimental import pallas as pl
from jax.experimental.pallas import tpu as pltpu

N_DEV = 4
B, Sq, Hq, Hkv, Dh = 2, 256, 8, 2, 64
G = Hq // Hkv
NB = B * Hkv
SCALE = 0.125


def kernel(x, Wq, Wo, K_ext, V_ext):
    c = K_ext.shape[1]
    R = G * Sq

    def body(x_ref, wq_ref, wo_ref, k_ref, v_ref, out_ref,
             s1buf, r1buf, s2buf, r2buf, ls1, lr1, ls2, lr2,
             s1_send, s1_recv, s2_send, s2_recv, l_send, l_recv):
        my = lax.axis_index("i")
        p1 = my ^ 1
        p2 = 3 - my

        barrier = pltpu.get_barrier_semaphore()
        for p in (p1, p2):
            pl.semaphore_signal(barrier, inc=1, device_id=(p,),
                                device_id_type=pl.DeviceIdType.MESH)
        pl.semaphore_wait(barrier, 2)

        qg = []
        for b in range(B):
            qb = jnp.dot(x_ref[b].astype(jnp.bfloat16),
                         wq_ref[...].astype(jnp.bfloat16),
                         preferred_element_type=jnp.float32)
            qbh = (qb * SCALE).astype(jnp.bfloat16)
            for kvh in range(Hkv):
                qg.append(jnp.concatenate(
                    [qbh[:, (kvh * G + g) * Dh:(kvh * G + g + 1) * Dh]
                     for g in range(G)], axis=0))

        def acc_rdma(src, dst, send_sems, recv_sems, idx, peer):
            return pltpu.make_async_remote_copy(
                src_ref=src.at[idx], dst_ref=dst.at[idx],
                send_sem=send_sems.at[idx], recv_sem=recv_sems.at[idx],
                device_id=(peer,), device_id_type=pl.DeviceIdType.MESH)

        acc0, l0, rd1 = [], [], []
        for b in range(B):
            for kvh in range(Hkv):
                idx = b * Hkv + kvh
                kc = k_ref[b][:, kvh, :].astype(jnp.bfloat16)
                vc = v_ref[b][:, kvh, :].astype(jnp.bfloat16)
                s = lax.dot_general(qg[idx], kc, (((1,), (1,)), ((), ())),
                                    preferred_element_type=jnp.float32)
                p = jnp.exp(s)
                l0.append(jnp.sum(p, axis=-1, keepdims=True))
                a = jnp.dot(p.astype(jnp.bfloat16), vc,
                            preferred_element_type=jnp.float32)
                acc0.append(a)
                s1buf[idx] = a.astype(jnp.bfloat16)
                r = acc_rdma(s1buf, r1buf, s1_send, s1_recv, idx, p1)
                r.start()
                rd1.append(r)
        for idx in range(NB):
            ls1[idx] = l0[idx]
        l1 = pltpu.make_async_remote_copy(
            src_ref=ls1, dst_ref=lr1, send_sem=l_send.at[0],
            recv_sem=l_recv.at[0], device_id=(p1,),
            device_id_type=pl.DeviceIdType.MESH)
        l1.start()

        acc1, rd2 = [], []
        for idx in range(NB):
            rd1[idx].wait_recv()
            a = acc0[idx] + r1buf[idx].astype(jnp.float32)
            acc1.append(a)
            s2buf[idx] = a.astype(jnp.bfloat16)
            r = acc_rdma(s2buf, r2buf, s2_send, s2_recv, idx, p2)
            r.start()
            rd2.append(r)
        l1.wait_recv()
        lsum1 = [l0[idx] + lr1[idx] for idx in range(NB)]
        for idx in range(NB):
            ls2[idx] = lsum1[idx]
        l2 = pltpu.make_async_remote_copy(
            src_ref=ls2, dst_ref=lr2, send_sem=l_send.at[1],
            recv_sem=l_recv.at[1], device_id=(p2,),
            device_id_type=pl.DeviceIdType.MESH)
        l2.start()

        acc2 = []
        for idx in range(NB):
            rd2[idx].wait_recv()
            acc2.append(acc1[idx] + r2buf[idx].astype(jnp.float32))
        l2.wait_recv()
        ltot = [lsum1[idx] + lr2[idx] for idx in range(NB)]

        wo_b = wo_ref[...].astype(jnp.bfloat16)
        for b in range(B):
            heads = []
            for h in range(Hq):
                kvh, g = h // G, h % G
                idx = b * Hkv + kvh
                o = (acc2[idx][g * Sq:(g + 1) * Sq, :]
                     / ltot[idx][g * Sq:(g + 1) * Sq, :])
                heads.append(o)
            ob = jnp.concatenate(heads, axis=1)
            out_ref[b] = jnp.dot(ob.astype(jnp.bfloat16), wo_b,
                                 preferred_element_type=jnp.float32)

        for idx in range(NB):
            rd1[idx].wait_send()
            rd2[idx].wait_send()
        l1.wait_send()
        l2.wait_send()

    return pl.pallas_call(
        body,
        out_shape=jax.ShapeDtypeStruct((B, Sq, Wo.shape[1]), jnp.float32),
        in_specs=[pl.BlockSpec(memory_space=pltpu.VMEM)] * 5,
        out_specs=pl.BlockSpec(memory_space=pltpu.VMEM),
        scratch_shapes=[
            pltpu.VMEM((NB, R, Dh), jnp.bfloat16),
            pltpu.VMEM((NB, R, Dh), jnp.bfloat16),
            pltpu.VMEM((NB, R, Dh), jnp.bfloat16),
            pltpu.VMEM((NB, R, Dh), jnp.bfloat16),
            pltpu.VMEM((NB, R, 1), jnp.float32),
            pltpu.VMEM((NB, R, 1), jnp.float32),
            pltpu.VMEM((NB, R, 1), jnp.float32),
            pltpu.VMEM((NB, R, 1), jnp.float32),
            pltpu.SemaphoreType.DMA((NB,)),
            pltpu.SemaphoreType.DMA((NB,)),
            pltpu.SemaphoreType.DMA((NB,)),
            pltpu.SemaphoreType.DMA((NB,)),
            pltpu.SemaphoreType.DMA((2,)),
            pltpu.SemaphoreType.DMA((2,)),
        ],
        compiler_params=pltpu.CompilerParams(collective_id=0),
    )(x, Wq, Wo, K_ext, V_ext)


# device time: 31189 ns/iter; 2.2897x vs baseline; 2.2897x over previous
import jax
import jax.numpy as jnp
from jax import lax
from jax.experimental import pallas as pl
from jax.experimental.pallas import tpu as pltpu

N_DEV = 4
B, Sq, Hq, Hkv, Dh = 2, 256, 8, 2, 64
G = Hq // Hkv
SCALE = 0.125


def kernel(x, Wq, Wo, K_ext, V_ext):
    c = K_ext.shape[1]

    def body(x_ref, wq_ref, wo_ref, k_ref, v_ref, out_ref,
             kbuf, vbuf, k_send, k_recv, v_send, v_recv):
        my = lax.axis_index("i")

        barrier = pltpu.get_barrier_semaphore()
        for d in range(1, N_DEV):
            pl.semaphore_signal(barrier, inc=1, device_id=((my + d) % N_DEV,),
                                device_id_type=pl.DeviceIdType.MESH)
        pl.semaphore_wait(barrier, N_DEV - 1)

        c2 = c // 2
        for b in range(B):
            kb = k_ref[b].astype(jnp.bfloat16)
            vb = v_ref[b].astype(jnp.bfloat16)
            for kvh in range(Hkv):
                for h in range(2):
                    sl = slice(h * c2, (h + 1) * c2)
                    kbuf[0, h, b, kvh] = jnp.transpose(kb[sl, kvh, :])
                    vbuf[0, h, b, kvh] = vb[sl, kvh, :]

        krd, vrd = {}, {}
        for d in (1, 3, 2):
            tgt = (my + d) % N_DEV
            for h in range(2):
                krd[d, h] = pltpu.make_async_remote_copy(
                    src_ref=kbuf.at[0, h], dst_ref=kbuf.at[N_DEV - d, h],
                    send_sem=k_send.at[(d - 1) * 2 + h],
                    recv_sem=k_recv.at[(N_DEV - 1 - d) * 2 + h],
                    device_id=(tgt,), device_id_type=pl.DeviceIdType.MESH)
                vrd[d, h] = pltpu.make_async_remote_copy(
                    src_ref=vbuf.at[0, h], dst_ref=vbuf.at[N_DEV - d, h],
                    send_sem=v_send.at[(d - 1) * 2 + h],
                    recv_sem=v_recv.at[(N_DEV - 1 - d) * 2 + h],
                    device_id=(tgt,), device_id_type=pl.DeviceIdType.MESH)
                krd[d, h].start()
                vrd[d, h].start()

        qg = []
        for b in range(B):
            qb = jnp.dot(x_ref[b].astype(jnp.bfloat16),
                         wq_ref[...].astype(jnp.bfloat16),
                         preferred_element_type=jnp.float32)
            qbh = (qb * SCALE).astype(jnp.bfloat16)
            qg.append([
                jnp.concatenate(
                    [qbh[:, (kvh * G + g) * Dh:(kvh * G + g + 1) * Dh]
                     for g in range(G)], axis=0)
                for kvh in range(Hkv)
            ])

        l = [[jnp.zeros((G * Sq, 1), jnp.float32)
              for _ in range(Hkv)] for _ in range(B)]
        acc = [[jnp.zeros((G * Sq, Dh), jnp.float32)
                for _ in range(Hkv)] for _ in range(B)]

        def fold_half(slot, h, k_wait=None, v_wait=None):
            if k_wait is not None:
                k_wait.wait_recv()
            ps = []
            for b in range(B):
                for kvh in range(Hkv):
                    kc = kbuf[slot, h, b, kvh]
                    s = jnp.dot(qg[b][kvh], kc,
                                preferred_element_type=jnp.float32)
                    p = jnp.exp(s)
                    l[b][kvh] = l[b][kvh] + jnp.sum(
                        p, axis=-1, keepdims=True)
                    ps.append(p.astype(jnp.bfloat16))
            if v_wait is not None:
                v_wait.wait_recv()
            for b in range(B):
                for kvh in range(Hkv):
                    vc = vbuf[slot, h, b, kvh]
                    acc[b][kvh] = acc[b][kvh] + jnp.dot(
                        ps[b * Hkv + kvh], vc,
                        preferred_element_type=jnp.float32)

        fold_half(0, 0)
        fold_half(0, 1)
        for slot, d in ((3, 1), (1, 3), (2, 2)):
            for h in range(2):
                fold_half(slot, h, k_wait=krd[d, h], v_wait=vrd[d, h])

        wo_b = wo_ref[...].astype(jnp.bfloat16)
        for b in range(B):
            heads = []
            for h in range(Hq):
                kvh, g = h // G, h % G
                o = (acc[b][kvh][g * Sq:(g + 1) * Sq, :]
                     / l[b][kvh][g * Sq:(g + 1) * Sq, :])
                heads.append(o)
            ob = jnp.concatenate(heads, axis=1)
            out_ref[b] = jnp.dot(ob.astype(jnp.bfloat16), wo_b,
                                 preferred_element_type=jnp.float32)

        for d in range(1, N_DEV):
            for h in range(2):
                krd[d, h].wait_send()
                vrd[d, h].wait_send()

    return pl.pallas_call(
        body,
        out_shape=jax.ShapeDtypeStruct((B, Sq, Wo.shape[1]), jnp.float32),
        in_specs=[pl.BlockSpec(memory_space=pltpu.VMEM)] * 5,
        out_specs=pl.BlockSpec(memory_space=pltpu.VMEM),
        scratch_shapes=[
            pltpu.VMEM((N_DEV, 2, B, Hkv, Dh, c // 2), jnp.bfloat16),
            pltpu.VMEM((N_DEV, 2, B, Hkv, c // 2, Dh), jnp.bfloat16),
            pltpu.SemaphoreType.DMA(((N_DEV - 1) * 2,)),
            pltpu.SemaphoreType.DMA(((N_DEV - 1) * 2,)),
            pltpu.SemaphoreType.DMA(((N_DEV - 1) * 2,)),
            pltpu.SemaphoreType.DMA(((N_DEV - 1) * 2,)),
        ],
        compiler_params=pltpu.CompilerParams(collective_id=0),
    )(x, Wq, Wo, K_ext, V_ext)
